# Initial kernel scaffold; baseline (speedup 1.0000x reference)
import jax
import jax.numpy as jnp
from jax.experimental import pallas as pl


def kernel(nodes, neigh_idx, features, weight):
    raise NotImplementedError("write your pallas kernel here")


# speedup vs baseline: 4.1031x; 4.1031x over previous
import functools

import jax
import jax.numpy as jnp
from jax import lax
from jax.experimental import pallas as pl
from jax.experimental.pallas import tpu as pltpu
from jax.experimental.pallas import tpu_sc as plsc

NC = 2
NS = 16
NW = NC * NS
C = 16
VL = 16


def _sc_gather_mean(neigh_flat, nodes_p, features, b_per_w, s):
    b_pad = nodes_p.shape[0]
    d = features.shape[1]
    rows = C * s
    n_chunks = b_per_w // C
    nvec = d // VL
    splits = []
    off = 0
    while off < rows:
        n = min(128, rows - off)
        splits.append((off, n))
        off += n

    mesh = plsc.VectorSubcoreMesh(core_axis_name="c", subcore_axis_name="s")

    def body(neigh_hbm, nodes_hbm, feat_hbm, self_out, agg_out,
             nidx, sidx, rows_v, selfr, agg, sem_n, sem_s):
        wid = lax.axis_index("s") * NC + lax.axis_index("c")
        base = wid * b_per_w

        def chunk(ci, carry):
            cb = base + ci * C
            pltpu.sync_copy(neigh_hbm.at[pl.ds(cb * s, rows)], nidx)
            pltpu.sync_copy(nodes_hbm.at[pl.ds(cb, C)], sidx)
            copies = [
                pltpu.async_copy(feat_hbm.at[nidx.at[pl.ds(o, n)]],
                                 rows_v.at[pl.ds(o, n)], sem_n)
                for (o, n) in splits
            ]
            self_cp = pltpu.async_copy(feat_hbm.at[sidx], selfr, sem_s)
            for cp in copies:
                cp.wait()

            def node(i, c2):
                def row(j, accs):
                    r = i * s + j
                    return tuple(accs[v] + rows_v[r, pl.ds(v * VL, VL)]
                                 for v in range(nvec))
                accs = lax.fori_loop(
                    0, s, row,
                    tuple(jnp.zeros((VL,), jnp.float32) for _ in range(nvec)))
                inv = jnp.float32(1.0 / s)
                for v in range(nvec):
                    agg[i, pl.ds(v * VL, VL)] = accs[v] * inv
                return c2

            lax.fori_loop(0, C, node, 0)
            pltpu.sync_copy(agg, agg_out.at[pl.ds(cb, C)])
            self_cp.wait()
            pltpu.sync_copy(selfr, self_out.at[pl.ds(cb, C)])
            return carry

        lax.fori_loop(0, n_chunks, chunk, 0)

    f = pl.kernel(
        body,
        out_type=(jax.ShapeDtypeStruct((b_pad, d), jnp.float32),
                  jax.ShapeDtypeStruct((b_pad, d), jnp.float32)),
        mesh=mesh,
        scratch_types=[
            pltpu.VMEM((rows,), jnp.int32),
            pltpu.VMEM((C,), jnp.int32),
            pltpu.VMEM((rows, d), jnp.float32),
            pltpu.VMEM((C, d), jnp.float32),
            pltpu.VMEM((C, d), jnp.float32),
            pltpu.SemaphoreType.DMA,
            pltpu.SemaphoreType.DMA,
        ],
    )
    return f(neigh_flat, nodes_p, features)


def _tc_matmul(selfs, aggs, weight, bt=512):
    b_pad, d = selfs.shape
    e = weight.shape[0]

    def body(self_ref, agg_ref, w_ref, out_ref):
        comb = jnp.concatenate([self_ref[...], agg_ref[...]], axis=1)
        acc = lax.dot_general(w_ref[...], comb, (((1,), (1,)), ((), ())),
                              preferred_element_type=jnp.float32)
        out_ref[...] = jnp.maximum(acc, 0.0)

    return pl.pallas_call(
        body,
        grid=(b_pad // bt,),
        in_specs=[
            pl.BlockSpec((bt, d), lambda i: (i, 0)),
            pl.BlockSpec((bt, d), lambda i: (i, 0)),
            pl.BlockSpec((e, 2 * d), lambda i: (0, 0)),
        ],
        out_specs=pl.BlockSpec((e, bt), lambda i: (0, i)),
        out_shape=jax.ShapeDtypeStruct((e, b_pad), jnp.float32),
    )(selfs, aggs, weight)


def kernel(nodes, neigh_idx, features, weight):
    b = nodes.shape[0]
    n, d = features.shape
    s = neigh_idx.shape[1]

    quantum = NW * C
    b_pad = -(-b // quantum) * quantum
    pad = b_pad - b
    if pad:
        pad_nodes = (jnp.arange(pad, dtype=jnp.int32) * 97) % n
        nodes_p = jnp.concatenate([nodes, pad_nodes])
        pad_neigh = ((jnp.arange(pad * s, dtype=jnp.int32) * 131) % n)
        neigh_p = jnp.concatenate([neigh_idx.reshape(-1), pad_neigh])
    else:
        nodes_p = nodes
        neigh_p = neigh_idx.reshape(-1)

    selfs, aggs = _sc_gather_mean(neigh_p, nodes_p, features, b_pad // NW, s)
    out = _tc_matmul(selfs, aggs, weight)
    return out[:, :b]
